# baseline (speedup 1.0000x reference)
import functools

import jax
import jax.numpy as jnp
from jax import lax
from jax.experimental import pallas as pl
from jax.experimental.pallas import tpu as pltpu
from jax.experimental.pallas import tpu_sc as plsc

B, A, C, K = 8, 50000, 80, 100
AP = 50048
CT = 8
NEG = float("-inf")


def _topk_body(s_ref, v_ref, i_ref):
    b = pl.program_id(0)
    v0 = s_ref[0]
    ia = lax.broadcasted_iota(jnp.int32, (CT, AP), 1)
    ik = lax.broadcasted_iota(jnp.int32, (CT, K), 1)

    def step(r, carry):
        v, accv, acci = carry
        m = jnp.max(v, axis=1, keepdims=True)
        p = jnp.min(jnp.where(v == m, ia, AP), axis=1, keepdims=True)
        accv = jnp.where(ik == r, m, accv)
        acci = jnp.where(ik == r, p, acci)
        v = jnp.where(ia == p, NEG, v)
        return v, accv, acci

    _, accv, acci = lax.fori_loop(
        0, K, step,
        (v0, jnp.zeros((CT, K), jnp.float32), jnp.zeros((CT, K), jnp.int32)),
    )
    v_ref[0] = accv
    i_ref[0] = acci + b * A


def _topk(s_pad):
    return pl.pallas_call(
        _topk_body,
        grid=(B, C // CT),
        in_specs=[pl.BlockSpec((1, CT, AP), lambda b, ct: (b, ct, 0))],
        out_specs=[
            pl.BlockSpec((1, CT, K), lambda b, ct: (b, ct, 0)),
            pl.BlockSpec((1, CT, K), lambda b, ct: (b, ct, 0)),
        ],
        out_shape=[
            jax.ShapeDtypeStruct((B, C, K), jnp.float32),
            jax.ShapeDtypeStruct((B, C, K), jnp.int32),
        ],
    )(s_pad)


N_IDX = B * K * C
D_PAD = 128
CH = 400


def _make_gather():
    info = plsc.get_sparse_core_info()
    nc, ns = info.num_cores, info.num_subcores
    nw = nc * ns
    b_per_w = N_IDX // nw
    n_ch = b_per_w // CH
    mesh = plsc.VectorSubcoreMesh(core_axis_name="c", subcore_axis_name="s")

    @functools.partial(
        pl.kernel, mesh=mesh,
        out_type=jax.ShapeDtypeStruct((N_IDX, D_PAD), jnp.float32),
        scratch_types=[
            pltpu.VMEM((CH,), jnp.int32),
            pltpu.VMEM((CH, D_PAD), jnp.float32),
            pltpu.SemaphoreType.DMA,
        ],
    )
    def gather_k(table_hbm, idx_hbm, out_hbm, idx_v, rows_v, sem):
        wid = lax.axis_index("s") * nc + lax.axis_index("c")
        base = wid * b_per_w
        for j in range(n_ch):
            pltpu.sync_copy(idx_hbm.at[pl.ds(base + j * CH, CH)], idx_v)
            pltpu.async_copy(table_hbm.at[idx_v], rows_v, sem).wait()
            pltpu.sync_copy(rows_v, out_hbm.at[pl.ds(base + j * CH, CH)])

    return gather_k


def kernel(scores, boxes):
    s2 = jnp.transpose(scores, (0, 2, 1))
    s2 = jnp.pad(s2, ((0, 0), (0, 0), (0, AP - A)), constant_values=-jnp.inf)
    vals_ck, idx_ck = _topk(s2)
    vals = jnp.transpose(vals_ck, (0, 2, 1))
    idxf = jnp.transpose(idx_ck, (0, 2, 1)).reshape(N_IDX)
    table = jnp.pad(boxes.reshape(B * A, 4), ((0, 0), (0, D_PAD - 4)))
    rows = _make_gather()(table, idxf)
    gboxes = rows[:, :4].reshape(B, K, C, 4)
    return (vals, gboxes)
